# baseline (speedup 1.0000x reference)
import functools

import jax
import jax.numpy as jnp
from jax import lax
from jax.experimental import pallas as pl
from jax.experimental.pallas import tpu as pltpu
from jax.experimental.pallas import tpu_sc as plsc

F = 26
VOC = 1000
D = 16
B = 4096

NC = 2
NS = 16
NW = NC * NS
NPAIR = (F * (F - 1)) // 2
PAIRS_PER_W = -(-NPAIR // NW)
NBC = B // D
SEG = B // NS
XW = SEG * F


def _decode_pair(p):
    def body(t, carry):
        rem, ii, act = carry
        rowlen = F - 1 - t
        take = jnp.logical_and(act == 1, rem >= rowlen)
        rem = jnp.where(take, rem - rowlen, rem)
        ii = jnp.where(take, ii + 1, ii)
        act = jnp.where(take, act, 0)
        return (rem, ii, act)

    rem, ii, _ = lax.fori_loop(0, F - 1, body, (p, 0, 1))
    return ii, ii + 1 + rem


def _ffm_sc_body(vd, w1d, xflat, out,
                 tbla0, tblb0, tbla1, tblb1, cola_v, colb_v, partial_v,
                 w1line_v, seg_v, segtmp_v, xwin_v, coltile_v,
                 xtsh, shared, sem0, sem1):
    scid = lax.axis_index("c")
    sid = lax.axis_index("s")
    wid = sid * NC + scid
    sems = (sem0, sem1)
    tbls = ((tbla0, tblb0), (tbla1, tblb1))

    pltpu.sync_copy(xflat.at[pl.ds(sid * XW, XW)], xwin_v)
    lanes = lax.iota(jnp.int32, D)
    lanesF = lanes * F
    for i in range(F):
        for c in range(SEG // D):
            idx16 = lanesF + (c * D * F + i)
            v = plsc.load_gather(xwin_v, [idx16])
            coltile_v[i, pl.ds(c * D, D)] = v
    pltpu.sync_copy(coltile_v, xtsh.at[:, pl.ds(sid * SEG, SEG)])

    zero16 = jnp.zeros((D,), jnp.float32)
    dsplats = [jnp.full((D,), d, jnp.int32) for d in range(D)]

    def zero_body(c, carry):
        partial_v[pl.ds(c * D, D)] = zero16
        return carry

    lax.fori_loop(0, B // D, zero_body, 0)
    plsc.subcore_barrier()

    @pl.when(wid < F)
    def _():
        f = wid
        pltpu.sync_copy(w1d.at[f], w1line_v)
        pltpu.sync_copy(xtsh.at[f], cola_v)

        def fo_body(bc, carry):
            xi = cola_v[pl.ds(bc * D, D)]
            w = plsc.load_gather(w1line_v, [xi])
            plsc.addupdate(partial_v.at[pl.ds(bc * D, D)], w)
            return carry

        lax.fori_loop(0, NBC, fo_body, 0)

    def start_fetch(k):
        p = wid + NW * k
        valid = p < NPAIR
        pc = jnp.minimum(p, NPAIR - 1)
        i, j = _decode_pair(pc)
        ta, tb = tbls[k % 2]
        cps = [
            pltpu.async_copy(
                vd.at[pl.ds((i * F + j) * D, D)], ta, sems[k % 2]),
            pltpu.async_copy(
                vd.at[pl.ds((j * F + i) * D, D)], tb, sems[k % 2]),
        ]
        return (i, j, valid, cps)

    inflight = start_fetch(0)
    for k in range(PAIRS_PER_W):
        i, j, valid, cps = inflight
        nxt = start_fetch(k + 1) if k + 1 < PAIRS_PER_W else None
        ta, tb = tbls[k % 2]

        for cp in cps:
            cp.wait()

        @pl.when(valid)
        def _():
            pltpu.sync_copy(xtsh.at[i], cola_v)
            pltpu.sync_copy(xtsh.at[j], colb_v)

            def pair_body(bc, carry):
                xi = cola_v[pl.ds(bc * D, D)]
                xj = colb_v[pl.ds(bc * D, D)]
                accs = [zero16, zero16, zero16, zero16]
                for d in range(D):
                    a = plsc.load_gather(ta, [dsplats[d], xi])
                    b = plsc.load_gather(tb, [dsplats[d], xj])
                    accs[d % 4] = accs[d % 4] + a * b
                plsc.addupdate(
                    partial_v.at[pl.ds(bc * D, D)],
                    (accs[0] + accs[1]) + (accs[2] + accs[3]))
                return carry

            lax.fori_loop(0, NBC, pair_body, 0)

        inflight = nxt

    pltpu.sync_copy(partial_v, shared.at[sid])
    plsc.subcore_barrier()

    def zseg_body(c, carry):
        seg_v[pl.ds(c * D, D)] = zero16
        return carry

    lax.fori_loop(0, SEG // D, zseg_body, 0)

    def red_body(t, carry):
        pltpu.sync_copy(shared.at[t, pl.ds(sid * SEG, SEG)], segtmp_v)
        for c in range(SEG // D):
            sl = pl.ds(c * D, D)
            seg_v[sl] = seg_v[sl] + segtmp_v[sl]
        return carry

    lax.fori_loop(0, NS, red_body, 0)
    pltpu.sync_copy(seg_v, out.at[pl.ds(scid * B + sid * SEG, SEG)])


@functools.cache
def _build_ffm_sc():
    return functools.partial(
        pl.kernel,
        out_type=jax.ShapeDtypeStruct((NC * B,), jnp.float32),
        mesh=plsc.VectorSubcoreMesh(
            core_axis_name="c", subcore_axis_name="s",
            num_cores=NC, num_subcores=NS),
        scratch_types=[
            pltpu.VMEM((D, VOC), jnp.float32),
            pltpu.VMEM((D, VOC), jnp.float32),
            pltpu.VMEM((D, VOC), jnp.float32),
            pltpu.VMEM((D, VOC), jnp.float32),
            pltpu.VMEM((B,), jnp.int32),
            pltpu.VMEM((B,), jnp.int32),
            pltpu.VMEM((B,), jnp.float32),
            pltpu.VMEM((VOC,), jnp.float32),
            pltpu.VMEM((SEG,), jnp.float32),
            pltpu.VMEM((SEG,), jnp.float32),
            pltpu.VMEM((XW,), jnp.int32),
            pltpu.VMEM((F, SEG), jnp.int32),
            pltpu.VMEM_SHARED((F, B), jnp.int32),
            pltpu.VMEM_SHARED((NS, B), jnp.float32),
            pltpu.SemaphoreType.DMA,
            pltpu.SemaphoreType.DMA,
        ],
        compiler_params=pltpu.CompilerParams(
            needs_layout_passes=False, use_tc_tiling_on_sc=True),
    )(_ffm_sc_body)


def kernel(x, W1, V, bias):
    vd = jnp.swapaxes(V, 2, 3).reshape(F * F * D, VOC)
    w1d = jnp.swapaxes(W1, 1, 2).reshape(F, VOC)
    xflat = x.astype(jnp.int32).reshape(B * F)

    out = _build_ffm_sc()(vd, w1d, xflat)
    return (out[:B] + out[B:]).reshape(B, 1) + bias
